# baseline (speedup 1.0000x reference)
import jax
import jax.numpy as jnp
from jax import lax
from jax.experimental import pallas as pl
from jax.experimental.pallas import tpu as pltpu
from jax.experimental.pallas import tpu_sc as plsc

BATCH = 8
N_DOFS = 262144
N_BOUNDARY = 8192
D = 32

NUM_CORES = 2
NUM_SUBCORES = 16

LANES = 128
SUBL = 8
DBLK = D // SUBL
CBLK_IN = N_DOFS // LANES
CBLK_OUT = N_BOUNDARY // LANES
WORDS_PER_BLK = CBLK_IN * SUBL * LANES
GRP = 1024
N_GRP = N_BOUNDARY // GRP


def _gather_body(xv_hbm, idx_hbm, out_hbm, idx_v, out_v, sem0, sem1):
    wid = lax.axis_index("s") * NUM_CORES + lax.axis_index("c")
    b = wid // DBLK
    t = wid % DBLK

    pltpu.sync_copy(idx_hbm, idx_v)

    block = xv_hbm.at[b, t]

    def _fire(g, _):
        for k in range(GRP // 16):
            v = idx_v[g, pl.ds(k * 16, 16)]
            idx_v[g, pl.ds(k * 16, 16)] = ((v >> 7) << 10) | (v & 127)
        rows = idx_v.at[g]
        for s in range(SUBL):
            table = block.at[pl.ds(s * LANES, WORDS_PER_BLK - SUBL * LANES + LANES)]
            pltpu.async_copy(table.at[rows], out_v.at[g, s], sem0)
        return _

    lax.fori_loop(0, N_GRP, _fire, 0)

    for _ in range(N_GRP * SUBL):
        pltpu.make_async_copy(block.at[pl.ds(0, GRP)], out_v.at[0, 0], sem0).wait()

    wb = []
    for g in range(N_GRP):
        for c8 in range(CBLK_OUT // N_GRP):
            wb.append(
                pltpu.async_copy(
                    out_v.at[g, :, pl.ds(c8 * LANES, LANES)],
                    out_hbm.at[b, t, g * (CBLK_OUT // N_GRP) + c8],
                    sem1,
                )
            )
    for cp in wb:
        cp.wait()


def _tile_view(x):
    xt = jnp.transpose(x, (0, 2, 1))
    x5 = xt.reshape(BATCH, DBLK, SUBL, CBLK_IN, LANES)
    x5 = jnp.transpose(x5, (0, 1, 3, 2, 4))
    return x5.reshape(BATCH, DBLK, CBLK_IN * SUBL * LANES)


def _untile_out(o):
    o5 = o.reshape(BATCH, DBLK, CBLK_OUT, SUBL, LANES)
    o5 = jnp.transpose(o5, (0, 1, 3, 2, 4))
    ot = o5.reshape(BATCH, D, N_BOUNDARY)
    return jnp.transpose(ot, (0, 2, 1))


@jax.jit
def kernel(x, indices):
    xv = _tile_view(x)
    idx2d = indices.reshape(N_GRP, GRP)
    run = pl.kernel(
        _gather_body,
        out_type=jax.ShapeDtypeStruct(
            (BATCH, DBLK, CBLK_OUT, SUBL, LANES), jnp.float32
        ),
        mesh=plsc.VectorSubcoreMesh(core_axis_name="c", subcore_axis_name="s"),
        scratch_types=[
            pltpu.VMEM((N_GRP, GRP), jnp.int32),
            pltpu.VMEM((N_GRP, SUBL, GRP), jnp.float32),
            pltpu.SemaphoreType.DMA,
            pltpu.SemaphoreType.DMA,
        ],
        compiler_params=pltpu.CompilerParams(use_tc_tiling_on_sc=False),
    )
    out = run(xv, idx2d)
    return _untile_out(out.reshape(BATCH, DBLK, CBLK_OUT * SUBL * LANES))


# speedup vs baseline: 1.0352x; 1.0352x over previous
import jax
import jax.numpy as jnp
from jax import lax
from jax.experimental import pallas as pl
from jax.experimental.pallas import tpu as pltpu
from jax.experimental.pallas import tpu_sc as plsc

BATCH = 8
N_DOFS = 262144
N_BOUNDARY = 8192
D = 32

NUM_CORES = 2
NUM_SUBCORES = 16

LANES = 128
SUBL = 8
DBLK = D // SUBL
CBLK_IN = N_DOFS // LANES
CBLK_OUT = N_BOUNDARY // LANES
WORDS_PER_BLK = CBLK_IN * SUBL * LANES
IDX_ROWS = N_BOUNDARY // 16
DMA_IDX = 8


def _gather_body(xv_hbm, idx_hbm, out_hbm, idx_v, out_v, sem0, sem1):
    wid = lax.axis_index("s") * NUM_CORES + lax.axis_index("c")
    b = wid // DBLK
    t = wid % DBLK

    pltpu.sync_copy(idx_hbm, idx_v)

    block = xv_hbm.at[b, t]
    half = CBLK_OUT // 2

    def _row(c, sem):
        for k in range(LANES // 16):
            v = idx_v[c, pl.ds(k * 16, 16)]
            idx_v[c, pl.ds(k * 16, 16)] = ((v >> 7) << 10) | (v & 127)
        rows = idx_v.at[c]
        for s in range(SUBL):
            table = block.at[pl.ds(s * LANES, WORDS_PER_BLK - SUBL * LANES + LANES)]
            pltpu.async_copy(table.at[rows], out_v.at[c, s], sem)

    def _fire0(c, _):
        _row(c, sem0)
        return _

    def _fire1(c, _):
        _row(c, sem1)
        return _

    lax.fori_loop(0, half, _fire0, 0)
    lax.fori_loop(half, CBLK_OUT, _fire1, 0)

    out_lo = out_hbm.at[b, t, pl.ds(0, half)]
    out_hi = out_hbm.at[b, t, pl.ds(half, half)]
    pltpu.make_async_copy(out_lo, out_v.at[pl.ds(0, half)], sem0).wait()
    pltpu.sync_copy(out_v.at[pl.ds(0, half)], out_lo)
    pltpu.make_async_copy(out_hi, out_v.at[pl.ds(half, half)], sem1).wait()
    pltpu.sync_copy(out_v.at[pl.ds(half, half)], out_hi)


def _tile_view(x):
    xt = jnp.transpose(x, (0, 2, 1))
    x5 = xt.reshape(BATCH, DBLK, SUBL, CBLK_IN, LANES)
    x5 = jnp.transpose(x5, (0, 1, 3, 2, 4))
    return x5.reshape(BATCH, DBLK, CBLK_IN * SUBL * LANES)


def _untile_out(o):
    o5 = o.reshape(BATCH, DBLK, CBLK_OUT, SUBL, LANES)
    o5 = jnp.transpose(o5, (0, 1, 3, 2, 4))
    ot = o5.reshape(BATCH, D, N_BOUNDARY)
    return jnp.transpose(ot, (0, 2, 1))


@jax.jit
def kernel(x, indices):
    xv = _tile_view(x)
    idx2d = indices.reshape(CBLK_OUT, LANES)
    run = pl.kernel(
        _gather_body,
        out_type=jax.ShapeDtypeStruct(
            (BATCH, DBLK, CBLK_OUT, SUBL, LANES), jnp.float32
        ),
        mesh=plsc.VectorSubcoreMesh(core_axis_name="c", subcore_axis_name="s"),
        scratch_types=[
            pltpu.VMEM((CBLK_OUT, LANES), jnp.int32),
            pltpu.VMEM((CBLK_OUT, SUBL, LANES), jnp.float32),
            pltpu.SemaphoreType.DMA,
            pltpu.SemaphoreType.DMA,
        ],
        compiler_params=pltpu.CompilerParams(use_tc_tiling_on_sc=False),
    )
    out = run(xv, idx2d)
    return _untile_out(out.reshape(BATCH, DBLK, CBLK_OUT * SUBL * LANES))
